# Initial kernel scaffold; baseline (speedup 1.0000x reference)
import functools

import jax
import jax.numpy as jnp
from jax import lax
from jax.experimental import pallas as pl
from jax.experimental.pallas import tpu as pltpu
from jax.experimental.pallas import tpu_sc as plsc

N = 10000
E = 320000
D = 128
H = 128
C = 64

NC = 2
NS = 16
NW = NC * NS
EPW = E // NW
CHUNK = 80
NCHUNK = EPW // CHUNK
RPS = N // NS

_MESH = plsc.VectorSubcoreMesh(core_axis_name="c", subcore_axis_name="s")


def _sc_agg_body(with_deg, *refs):
    if with_deg:
        (y_hbm, src_hbm, dst_hbm, z_hbm, z16_hbm, part_hbm, deg_hbm,
         srcv, dstv, rows, ones, acc, dacc, sem) = refs
    else:
        (y_hbm, src_hbm, dst_hbm, z_hbm, part_hbm,
         srcv, dstv, rows, acc, sem) = refs
    cid = lax.axis_index("c")
    sid = lax.axis_index("s")
    wid = cid * NS + sid
    r0 = sid * RPS

    pltpu.sync_copy(z_hbm.at[pl.ds(r0, RPS)], acc.at[pl.ds(r0, RPS)])
    if with_deg:
        pltpu.sync_copy(z16_hbm.at[pl.ds(r0, RPS)], dacc.at[pl.ds(r0, RPS)])

        @pl.loop(0, CHUNK)
        def _(i):
            ones[i] = jnp.full((16,), 1.0, jnp.float32)

    plsc.subcore_barrier()

    e_base = wid * EPW

    @pl.loop(0, NCHUNK)
    def _(c):
        e0 = e_base + c * CHUNK
        pltpu.sync_copy(src_hbm.at[pl.ds(e0, CHUNK)], srcv)
        pltpu.sync_copy(dst_hbm.at[pl.ds(e0, CHUNK)], dstv)
        pltpu.async_copy(y_hbm.at[srcv], rows, sem).wait()
        pltpu.sync_copy(rows, acc.at[dstv], add=True)
        if with_deg:
            pltpu.sync_copy(ones, dacc.at[dstv], add=True)

    plsc.subcore_barrier()
    pltpu.sync_copy(acc.at[pl.ds(r0, RPS)], part_hbm.at[cid, pl.ds(r0, RPS)])
    if with_deg:
        pltpu.sync_copy(dacc.at[pl.ds(r0, RPS)], deg_hbm.at[cid, pl.ds(r0, RPS)])


def _make_sc_agg(with_deg):
    out_type = [jax.ShapeDtypeStruct((NC, N, H), jnp.float32)]
    scratch = [
        pltpu.VMEM((CHUNK,), jnp.int32),
        pltpu.VMEM((CHUNK,), jnp.int32),
        pltpu.VMEM((CHUNK, H), jnp.float32),
    ]
    if with_deg:
        out_type.append(jax.ShapeDtypeStruct((NC, N, 16), jnp.float32))
        scratch.append(pltpu.VMEM((CHUNK, 16), jnp.float32))
    scratch.append(pltpu.VMEM_SHARED((N, H), jnp.float32))
    if with_deg:
        scratch.append(pltpu.VMEM_SHARED((N, 16), jnp.float32))
    scratch.append(pltpu.SemaphoreType.DMA)
    return pl.kernel(
        functools.partial(_sc_agg_body, with_deg),
        out_type=out_type,
        mesh=_MESH,
        scratch_types=scratch,
    )


_sc_agg_deg = _make_sc_agg(True)
_sc_agg = _make_sc_agg(False)

BN = 1000


def _mm_body(x_ref, w_ref, o_ref):
    o_ref[...] = jnp.dot(x_ref[...], w_ref[...],
                         preferred_element_type=jnp.float32)


def _tc_matmul(x, w):
    n, d = x.shape
    h = w.shape[1]
    return pl.pallas_call(
        _mm_body,
        grid=(n // BN,),
        in_specs=[pl.BlockSpec((BN, d), lambda i: (i, 0)),
                  pl.BlockSpec((d, h), lambda i: (0, 0))],
        out_specs=pl.BlockSpec((BN, h), lambda i: (i, 0)),
        out_shape=jax.ShapeDtypeStruct((n, h), jnp.float32),
    )(x, w)


def _layer_body(x_ref, ws_ref, b_ref, p_ref, d_ref, wn2_ref, h_ref, y2_ref):
    deg = d_ref[0, :, 0:1] + d_ref[1, :, 0:1]
    dinv = 1.0 / jnp.maximum(deg, 1.0)
    agg = (p_ref[0] + p_ref[1]) * dinv
    h = jnp.dot(x_ref[...], ws_ref[...], preferred_element_type=jnp.float32)
    h = jnp.maximum(h + agg + b_ref[...], 0.0)
    h_ref[...] = h
    y2_ref[...] = jnp.dot(h, wn2_ref[...], preferred_element_type=jnp.float32)


def _tc_layer(x, w_self, b, part, deg16, w_neigh2):
    return pl.pallas_call(
        _layer_body,
        grid=(N // BN,),
        in_specs=[pl.BlockSpec((BN, D), lambda i: (i, 0)),
                  pl.BlockSpec((D, H), lambda i: (0, 0)),
                  pl.BlockSpec((1, H), lambda i: (0, 0)),
                  pl.BlockSpec((NC, BN, H), lambda i: (0, i, 0)),
                  pl.BlockSpec((NC, BN, 16), lambda i: (0, i, 0)),
                  pl.BlockSpec((H, H), lambda i: (0, 0))],
        out_specs=[pl.BlockSpec((BN, H), lambda i: (i, 0)),
                   pl.BlockSpec((BN, H), lambda i: (i, 0))],
        out_shape=[jax.ShapeDtypeStruct((N, H), jnp.float32),
                   jax.ShapeDtypeStruct((N, H), jnp.float32)],
    )(x, w_self, b, part, deg16, w_neigh2)


def _final_body(h_ref, ws_ref, b_ref, q_ref, d_ref, wo_ref, bo_ref, o_ref):
    deg = d_ref[0, :, 0:1] + d_ref[1, :, 0:1]
    dinv = 1.0 / jnp.maximum(deg, 1.0)
    agg = (q_ref[0] + q_ref[1]) * dinv
    h2 = jnp.dot(h_ref[...], ws_ref[...], preferred_element_type=jnp.float32)
    h2 = jnp.maximum(h2 + agg + b_ref[...], 0.0)
    o_ref[...] = jnp.dot(h2, wo_ref[...],
                         preferred_element_type=jnp.float32) + bo_ref[...]


def _tc_final(h1, w_self2, b2, part, deg16, w_out, b_out):
    return pl.pallas_call(
        _final_body,
        grid=(N // BN,),
        in_specs=[pl.BlockSpec((BN, H), lambda i: (i, 0)),
                  pl.BlockSpec((H, H), lambda i: (0, 0)),
                  pl.BlockSpec((1, H), lambda i: (0, 0)),
                  pl.BlockSpec((NC, BN, H), lambda i: (0, i, 0)),
                  pl.BlockSpec((NC, BN, 16), lambda i: (0, i, 0)),
                  pl.BlockSpec((H, C), lambda i: (0, 0)),
                  pl.BlockSpec((1, C), lambda i: (0, 0))],
        out_specs=pl.BlockSpec((BN, C), lambda i: (i, 0)),
        out_shape=jax.ShapeDtypeStruct((N, C), jnp.float32),
    )(h1, w_self2, b2, part, deg16, w_out, b_out)


def kernel(features, edge_index, W_self1, W_neigh1, b1,
           W_self2, W_neigh2, b2, W_out, b_out):
    src = edge_index[0]
    dst = edge_index[1]
    z128 = jnp.zeros((N, H), jnp.float32)
    z16 = jnp.zeros((N, 16), jnp.float32)

    y1 = _tc_matmul(features, W_neigh1)
    part1, deg16 = _sc_agg_deg(y1, src, dst, z128, z16)
    h1, y2 = _tc_layer(features, W_self1, b1.reshape(1, H), part1, deg16,
                       W_neigh2)
    (part2,) = _sc_agg(y2, src, dst, z128)
    out = _tc_final(h1, W_self2, b2.reshape(1, H), part2, deg16,
                    W_out, b_out.reshape(1, C))
    return out


# speedup vs baseline: 3.4224x; 3.4224x over previous
import functools

import jax
import jax.numpy as jnp
from jax import lax
from jax.experimental import pallas as pl
from jax.experimental.pallas import tpu as pltpu
from jax.experimental.pallas import tpu_sc as plsc

N = 10000
E = 320000
D = 128
H = 128
C = 64

NC = 2
NS = 16
NW = NC * NS
EPW = E // NW
CHUNK = 80
NCHUNK = EPW // CHUNK
NA = 10240
RPS = NA // NS

_MESH = plsc.VectorSubcoreMesh(core_axis_name="c", subcore_axis_name="s")


def _sc_agg_body(with_deg, *refs):
    if with_deg:
        (y_hbm, src_hbm, dst_hbm, z_hbm, z16_hbm, part_hbm, deg_hbm,
         srcv, dstv, rows, ones, acc, dacc, sem) = refs
    else:
        (y_hbm, src_hbm, dst_hbm, z_hbm, part_hbm,
         srcv, dstv, rows, acc, sem) = refs
    cid = lax.axis_index("c")
    sid = lax.axis_index("s")
    wid = cid * NS + sid
    r0 = sid * RPS

    pltpu.sync_copy(z_hbm.at[pl.ds(r0, RPS)], acc.at[pl.ds(r0, RPS)])
    if with_deg:
        pltpu.sync_copy(z16_hbm.at[pl.ds(r0, RPS)], dacc.at[pl.ds(r0, RPS)])

        @pl.loop(0, CHUNK)
        def _(i):
            ones[i] = jnp.full((16,), 1.0, jnp.float32)

    plsc.subcore_barrier()

    e_base = wid * EPW

    @pl.loop(0, NCHUNK)
    def _(c):
        e0 = e_base + c * CHUNK
        pltpu.sync_copy(src_hbm.at[pl.ds(e0, CHUNK)], srcv)
        pltpu.sync_copy(dst_hbm.at[pl.ds(e0, CHUNK)], dstv)
        pltpu.async_copy(y_hbm.at[srcv], rows, sem).wait()
        pltpu.sync_copy(rows, acc.at[dstv], add=True)
        if with_deg:
            pltpu.sync_copy(ones, dacc.at[dstv], add=True)

    plsc.subcore_barrier()
    pltpu.sync_copy(acc.at[pl.ds(r0, RPS)], part_hbm.at[cid, pl.ds(r0, RPS)])
    if with_deg:
        pltpu.sync_copy(dacc.at[pl.ds(r0, RPS)], deg_hbm.at[cid, pl.ds(r0, RPS)])


def _make_sc_agg(with_deg):
    out_type = [jax.ShapeDtypeStruct((NC, NA, H), jnp.float32)]
    scratch = [
        pltpu.VMEM((CHUNK,), jnp.int32),
        pltpu.VMEM((CHUNK,), jnp.int32),
        pltpu.VMEM((CHUNK, H), jnp.float32),
    ]
    if with_deg:
        out_type.append(jax.ShapeDtypeStruct((NC, NA, 16), jnp.float32))
        scratch.append(pltpu.VMEM((CHUNK, 16), jnp.float32))
    scratch.append(pltpu.VMEM_SHARED((NA, H), jnp.float32))
    if with_deg:
        scratch.append(pltpu.VMEM_SHARED((NA, 16), jnp.float32))
    scratch.append(pltpu.SemaphoreType.DMA)
    return pl.kernel(
        functools.partial(_sc_agg_body, with_deg),
        out_type=out_type,
        mesh=_MESH,
        scratch_types=scratch,
    )


_sc_agg_deg = _make_sc_agg(True)
_sc_agg = _make_sc_agg(False)

BN = 1000


def _mm_body(x_ref, w_ref, o_ref):
    o_ref[...] = jnp.dot(x_ref[...], w_ref[...],
                         preferred_element_type=jnp.float32)


def _tc_matmul(x, w):
    n, d = x.shape
    h = w.shape[1]
    return pl.pallas_call(
        _mm_body,
        grid=(n // BN,),
        in_specs=[pl.BlockSpec((BN, d), lambda i: (i, 0)),
                  pl.BlockSpec((d, h), lambda i: (0, 0))],
        out_specs=pl.BlockSpec((BN, h), lambda i: (i, 0)),
        out_shape=jax.ShapeDtypeStruct((n, h), jnp.float32),
    )(x, w)


def _layer_body(x_ref, ws_ref, b_ref, p_ref, d_ref, wn2_ref, h_ref, y2_ref):
    deg = d_ref[0, :, 0:1] + d_ref[1, :, 0:1]
    dinv = 1.0 / jnp.maximum(deg, 1.0)
    agg = (p_ref[0] + p_ref[1]) * dinv
    h = jnp.dot(x_ref[...], ws_ref[...], preferred_element_type=jnp.float32)
    h = jnp.maximum(h + agg + b_ref[...], 0.0)
    h_ref[...] = h
    y2_ref[...] = jnp.dot(h, wn2_ref[...], preferred_element_type=jnp.float32)


def _tc_layer(x, w_self, b, part, deg16, w_neigh2):
    return pl.pallas_call(
        _layer_body,
        grid=(N // BN,),
        in_specs=[pl.BlockSpec((BN, D), lambda i: (i, 0)),
                  pl.BlockSpec((D, H), lambda i: (0, 0)),
                  pl.BlockSpec((1, H), lambda i: (0, 0)),
                  pl.BlockSpec((NC, BN, H), lambda i: (0, i, 0)),
                  pl.BlockSpec((NC, BN, 16), lambda i: (0, i, 0)),
                  pl.BlockSpec((H, H), lambda i: (0, 0))],
        out_specs=[pl.BlockSpec((BN, H), lambda i: (i, 0)),
                   pl.BlockSpec((BN, H), lambda i: (i, 0))],
        out_shape=[jax.ShapeDtypeStruct((N, H), jnp.float32),
                   jax.ShapeDtypeStruct((N, H), jnp.float32)],
    )(x, w_self, b, part, deg16, w_neigh2)


def _final_body(h_ref, ws_ref, b_ref, q_ref, d_ref, wo_ref, bo_ref, o_ref):
    deg = d_ref[0, :, 0:1] + d_ref[1, :, 0:1]
    dinv = 1.0 / jnp.maximum(deg, 1.0)
    agg = (q_ref[0] + q_ref[1]) * dinv
    h2 = jnp.dot(h_ref[...], ws_ref[...], preferred_element_type=jnp.float32)
    h2 = jnp.maximum(h2 + agg + b_ref[...], 0.0)
    o_ref[...] = jnp.dot(h2, wo_ref[...],
                         preferred_element_type=jnp.float32) + bo_ref[...]


def _tc_final(h1, w_self2, b2, part, deg16, w_out, b_out):
    return pl.pallas_call(
        _final_body,
        grid=(N // BN,),
        in_specs=[pl.BlockSpec((BN, H), lambda i: (i, 0)),
                  pl.BlockSpec((H, H), lambda i: (0, 0)),
                  pl.BlockSpec((1, H), lambda i: (0, 0)),
                  pl.BlockSpec((NC, BN, H), lambda i: (0, i, 0)),
                  pl.BlockSpec((NC, BN, 16), lambda i: (0, i, 0)),
                  pl.BlockSpec((H, C), lambda i: (0, 0)),
                  pl.BlockSpec((1, C), lambda i: (0, 0))],
        out_specs=pl.BlockSpec((BN, C), lambda i: (i, 0)),
        out_shape=jax.ShapeDtypeStruct((N, C), jnp.float32),
    )(h1, w_self2, b2, part, deg16, w_out, b_out)


def kernel(features, edge_index, W_self1, W_neigh1, b1,
           W_self2, W_neigh2, b2, W_out, b_out):
    src = edge_index[0]
    dst = edge_index[1]
    z128 = jnp.zeros((NA, H), jnp.float32)
    z16 = jnp.zeros((NA, 16), jnp.float32)

    y1 = _tc_matmul(features, W_neigh1)
    ones_mat = jnp.ones((N, H), jnp.float32)
    (deg128,) = _sc_agg(ones_mat, src, dst, z128)
    deg16 = deg128[:, :N, :16]
    (part1,) = _sc_agg(y1, src, dst, z128)
    part1 = part1[:, :N]
    h1, y2 = _tc_layer(features, W_self1, b1.reshape(1, H), part1, deg16,
                       W_neigh2)
    (part2,) = _sc_agg(y2, src, dst, z128)
    part2 = part2[:, :N]
    out = _tc_final(h1, W_self2, b2.reshape(1, H), part2, deg16,
                    W_out, b_out.reshape(1, C))
    return out


# speedup vs baseline: 8.1748x; 2.3886x over previous
import functools

import jax
import jax.numpy as jnp
from jax import lax
from jax.experimental import pallas as pl
from jax.experimental.pallas import tpu as pltpu
from jax.experimental.pallas import tpu_sc as plsc

N = 10000
E = 320000
D = 128
H = 128
C = 64

NC = 2
NS = 16
NW = NC * NS
EPW = E // NW
CHUNK = 80
NCHUNK = EPW // CHUNK
NA = 10240
RPS = NA // NS

_MESH = plsc.VectorSubcoreMesh(core_axis_name="c", subcore_axis_name="s")


SB = 25
NSB = NCHUNK // SB


def _sc_agg_body(*refs):
    (y_hbm, src_hbm, dst_hbm, z_hbm, part_hbm,
     srcv2, dstv2, rows0, rows1, acc, sem0, sem1) = refs
    cid = lax.axis_index("c")
    sid = lax.axis_index("s")
    wid = cid * NS + sid
    r0 = sid * RPS

    pltpu.sync_copy(z_hbm.at[pl.ds(r0, RPS)], acc.at[pl.ds(r0, RPS)])
    plsc.subcore_barrier()

    @pl.loop(0, NSB)
    def _(sb):
        pltpu.sync_copy(src_hbm.at[wid, sb], srcv2)
        pltpu.sync_copy(dst_hbm.at[wid, sb], dstv2)
        pltpu.async_copy(y_hbm.at[srcv2.at[0]], rows0, sem0)

        @pl.loop(0, SB // 2)
        def _(k):
            c0 = 2 * k
            c1 = c0 + 1
            g1 = pltpu.async_copy(y_hbm.at[srcv2.at[c1]], rows1, sem1)
            pltpu.make_async_copy(y_hbm.at[srcv2.at[c0]], rows0, sem0).wait()
            pltpu.sync_copy(rows0, acc.at[dstv2.at[c0]], add=True)
            pltpu.async_copy(y_hbm.at[srcv2.at[c0 + 2]], rows0, sem0)
            g1.wait()
            pltpu.sync_copy(rows1, acc.at[dstv2.at[c1]], add=True)

        pltpu.make_async_copy(y_hbm.at[srcv2.at[SB - 1]], rows0, sem0).wait()
        pltpu.sync_copy(rows0, acc.at[dstv2.at[SB - 1]], add=True)

    plsc.subcore_barrier()
    pltpu.sync_copy(acc.at[pl.ds(r0, RPS)], part_hbm.at[cid, pl.ds(r0, RPS)])


def _deg_body(*refs):
    (dst_hbm, ones_hbm, z_hbm, deg_hbm,
     dstv2, onesv, acc, sem) = refs
    cid = lax.axis_index("c")
    sid = lax.axis_index("s")
    wid = cid * NS + sid
    r0 = sid * RPS

    pltpu.sync_copy(z_hbm.at[pl.ds(r0, RPS)], acc.at[pl.ds(r0, RPS)])
    pltpu.async_copy(ones_hbm, onesv, sem).wait()
    plsc.subcore_barrier()

    @pl.loop(0, NSB)
    def _(sb):
        pltpu.sync_copy(dst_hbm.at[wid, sb], dstv2)

        @pl.loop(0, SB)
        def _(c):
            pltpu.sync_copy(onesv, acc.at[dstv2.at[c]], add=True)

    plsc.subcore_barrier()
    pltpu.sync_copy(acc.at[pl.ds(r0, RPS)], deg_hbm.at[cid, pl.ds(r0, RPS)])


_sc_agg = pl.kernel(
    _sc_agg_body,
    out_type=[jax.ShapeDtypeStruct((NC, NA, H), jnp.float32)],
    mesh=_MESH,
    scratch_types=[
        pltpu.VMEM((SB, CHUNK), jnp.int32),
        pltpu.VMEM((SB, CHUNK), jnp.int32),
        pltpu.VMEM((CHUNK, H), jnp.float32),
        pltpu.VMEM((CHUNK, H), jnp.float32),
        pltpu.VMEM_SHARED((NA, H), jnp.float32),
        pltpu.SemaphoreType.DMA,
        pltpu.SemaphoreType.DMA,
    ],
)

_sc_deg = pl.kernel(
    _deg_body,
    out_type=[jax.ShapeDtypeStruct((NC, NA, H), jnp.float32)],
    mesh=_MESH,
    scratch_types=[
        pltpu.VMEM((SB, CHUNK), jnp.int32),
        pltpu.VMEM((CHUNK, H), jnp.float32),
        pltpu.VMEM_SHARED((NA, H), jnp.float32),
        pltpu.SemaphoreType.DMA,
    ],
)

BN = 1000


def _mm_body(x_ref, w_ref, o_ref):
    o_ref[...] = jnp.dot(x_ref[...], w_ref[...],
                         preferred_element_type=jnp.float32)


def _tc_matmul(x, w):
    n, d = x.shape
    h = w.shape[1]
    return pl.pallas_call(
        _mm_body,
        grid=(n // BN,),
        in_specs=[pl.BlockSpec((BN, d), lambda i: (i, 0)),
                  pl.BlockSpec((d, h), lambda i: (0, 0))],
        out_specs=pl.BlockSpec((BN, h), lambda i: (i, 0)),
        out_shape=jax.ShapeDtypeStruct((n, h), jnp.float32),
    )(x, w)


def _layer_body(x_ref, ws_ref, b_ref, p_ref, d_ref, wn2_ref, h_ref, y2_ref):
    deg = d_ref[0, :, 0:1] + d_ref[1, :, 0:1]
    dinv = 1.0 / jnp.maximum(deg, 1.0)
    agg = (p_ref[0] + p_ref[1]) * dinv
    h = jnp.dot(x_ref[...], ws_ref[...], preferred_element_type=jnp.float32)
    h = jnp.maximum(h + agg + b_ref[...], 0.0)
    h_ref[...] = h
    y2_ref[...] = jnp.dot(h, wn2_ref[...], preferred_element_type=jnp.float32)


def _tc_layer(x, w_self, b, part, deg16, w_neigh2):
    return pl.pallas_call(
        _layer_body,
        grid=(N // BN,),
        in_specs=[pl.BlockSpec((BN, D), lambda i: (i, 0)),
                  pl.BlockSpec((D, H), lambda i: (0, 0)),
                  pl.BlockSpec((1, H), lambda i: (0, 0)),
                  pl.BlockSpec((NC, BN, H), lambda i: (0, i, 0)),
                  pl.BlockSpec((NC, BN, 16), lambda i: (0, i, 0)),
                  pl.BlockSpec((H, H), lambda i: (0, 0))],
        out_specs=[pl.BlockSpec((BN, H), lambda i: (i, 0)),
                   pl.BlockSpec((BN, H), lambda i: (i, 0))],
        out_shape=[jax.ShapeDtypeStruct((N, H), jnp.float32),
                   jax.ShapeDtypeStruct((N, H), jnp.float32)],
    )(x, w_self, b, part, deg16, w_neigh2)


def _final_body(h_ref, ws_ref, b_ref, q_ref, d_ref, wo_ref, bo_ref, o_ref):
    deg = d_ref[0, :, 0:1] + d_ref[1, :, 0:1]
    dinv = 1.0 / jnp.maximum(deg, 1.0)
    agg = (q_ref[0] + q_ref[1]) * dinv
    h2 = jnp.dot(h_ref[...], ws_ref[...], preferred_element_type=jnp.float32)
    h2 = jnp.maximum(h2 + agg + b_ref[...], 0.0)
    o_ref[...] = jnp.dot(h2, wo_ref[...],
                         preferred_element_type=jnp.float32) + bo_ref[...]


def _tc_final(h1, w_self2, b2, part, deg16, w_out, b_out):
    return pl.pallas_call(
        _final_body,
        grid=(N // BN,),
        in_specs=[pl.BlockSpec((BN, H), lambda i: (i, 0)),
                  pl.BlockSpec((H, H), lambda i: (0, 0)),
                  pl.BlockSpec((1, H), lambda i: (0, 0)),
                  pl.BlockSpec((NC, BN, H), lambda i: (0, i, 0)),
                  pl.BlockSpec((NC, BN, 16), lambda i: (0, i, 0)),
                  pl.BlockSpec((H, C), lambda i: (0, 0)),
                  pl.BlockSpec((1, C), lambda i: (0, 0))],
        out_specs=pl.BlockSpec((BN, C), lambda i: (i, 0)),
        out_shape=jax.ShapeDtypeStruct((N, C), jnp.float32),
    )(h1, w_self2, b2, part, deg16, w_out, b_out)


def kernel(features, edge_index, W_self1, W_neigh1, b1,
           W_self2, W_neigh2, b2, W_out, b_out):
    src3 = edge_index[0].reshape(NW, NSB, SB, CHUNK)
    dst3 = edge_index[1].reshape(NW, NSB, SB, CHUNK)
    z128 = jnp.zeros((NA, H), jnp.float32)
    ones_blk = jnp.ones((CHUNK, H), jnp.float32)

    y1 = _tc_matmul(features, W_neigh1)
    (deg128,) = _sc_deg(dst3, ones_blk, z128)
    deg16 = deg128[:, :N, :16]
    (part1,) = _sc_agg(y1, src3, dst3, z128)
    part1 = part1[:, :N]
    h1, y2 = _tc_layer(features, W_self1, b1.reshape(1, H), part1, deg16,
                       W_neigh2)
    (part2,) = _sc_agg(y2, src3, dst3, z128)
    part2 = part2[:, :N]
    out = _tc_final(h1, W_self2, b2.reshape(1, H), part2, deg16,
                    W_out, b_out.reshape(1, C))
    return out


# speedup vs baseline: 10.0658x; 1.2313x over previous
import functools

import jax
import jax.numpy as jnp
from jax import lax
from jax.experimental import pallas as pl
from jax.experimental.pallas import tpu as pltpu
from jax.experimental.pallas import tpu_sc as plsc

N = 10000
E = 320000
D = 128
H = 128
C = 64

NC = 2
NS = 16
NW = NC * NS
EPW = E // NW
CHUNK = 80
NCHUNK = EPW // CHUNK
SB = 25
NSB = NCHUNK // SB
NA = 10240
RPS = NA // NS

_MESH = plsc.VectorSubcoreMesh(core_axis_name="c", subcore_axis_name="s")


def _sc_agg_body(with_deg, *refs):
    if with_deg:
        (y_hbm, src_hbm, dst_hbm, z_hbm, z1_hbm, part_hbm, deg_hbm,
         srcv2, dstv2, rows0, rows1, ones1, acc, dacc,
         sg0, sg1, ss0, ss1) = refs
    else:
        (y_hbm, src_hbm, dst_hbm, z_hbm, part_hbm,
         srcv2, dstv2, rows0, rows1, acc,
         sg0, sg1, ss0, ss1) = refs
    cid = lax.axis_index("c")
    sid = lax.axis_index("s")
    wid = cid * NS + sid
    r0 = sid * RPS

    pltpu.sync_copy(z_hbm.at[pl.ds(r0, RPS)], acc.at[pl.ds(r0, RPS)])
    if with_deg:
        pltpu.sync_copy(z1_hbm.at[pl.ds(r0, RPS)], dacc.at[pl.ds(r0, RPS)])

        @pl.loop(0, CHUNK // 16)
        def _(i):
            ones1[pl.ds(i * 16, 16)] = jnp.full((16,), 1.0, jnp.float32)

    plsc.subcore_barrier()

    def gather(c, rows, sem):
        return pltpu.async_copy(y_hbm.at[srcv2.at[c]], rows, sem)

    def wait_gather(c, rows, sem):
        pltpu.make_async_copy(y_hbm.at[srcv2.at[c]], rows, sem).wait()

    def scatter(c, rows, sem):
        return pltpu.async_copy(rows, acc.at[dstv2.at[c]], sem, add=True)

    def wait_scatter(c, rows, sem):
        pltpu.make_async_copy(rows, acc.at[dstv2.at[c]], sem).wait()

    def deg_scatter(c):
        if with_deg:
            pltpu.sync_copy(ones1, dacc.at[dstv2.at[c]], add=True)

    def pair(c0, first):
        c1 = c0 + 1
        if not first:
            wait_scatter(c1, rows1, ss1)
        g1 = gather(c1, rows1, sg1)
        wait_gather(c0, rows0, sg0)
        scatter(c0, rows0, ss0)
        deg_scatter(c0)
        g1.wait()
        wait_scatter(c0, rows0, ss0)
        gather(c0 + 2, rows0, sg0)
        scatter(c1, rows1, ss1)
        deg_scatter(c1)

    @pl.loop(0, NSB)
    def _(sb):
        pltpu.sync_copy(src_hbm.at[wid, sb], srcv2)
        pltpu.sync_copy(dst_hbm.at[wid, sb], dstv2)
        gather(0, rows0, sg0)
        pair(0, first=True)

        @pl.loop(1, SB // 2)
        def _(k):
            pair(2 * k, first=False)

        wait_scatter(SB - 2, rows1, ss1)
        wait_gather(SB - 1, rows0, sg0)
        pltpu.sync_copy(rows0, acc.at[dstv2.at[SB - 1]], add=True)
        deg_scatter(SB - 1)

    plsc.subcore_barrier()
    pltpu.sync_copy(acc.at[pl.ds(r0, RPS)], part_hbm.at[cid, pl.ds(r0, RPS)])
    if with_deg:
        pltpu.sync_copy(dacc.at[pl.ds(r0, RPS)],
                        deg_hbm.at[cid, pl.ds(r0, RPS)])


def _make_sc_agg(with_deg):
    out_type = [jax.ShapeDtypeStruct((NC, NA, H), jnp.float32)]
    scratch = [
        pltpu.VMEM((SB, CHUNK), jnp.int32),
        pltpu.VMEM((SB, CHUNK), jnp.int32),
        pltpu.VMEM((CHUNK, H), jnp.float32),
        pltpu.VMEM((CHUNK, H), jnp.float32),
    ]
    if with_deg:
        out_type.append(jax.ShapeDtypeStruct((NC, NA), jnp.float32))
        scratch.append(pltpu.VMEM((CHUNK,), jnp.float32))
    scratch.append(pltpu.VMEM_SHARED((NA, H), jnp.float32))
    if with_deg:
        scratch.append(pltpu.VMEM_SHARED((NA,), jnp.float32))
    scratch += [pltpu.SemaphoreType.DMA] * 4
    return pl.kernel(
        functools.partial(_sc_agg_body, with_deg),
        out_type=out_type,
        mesh=_MESH,
        scratch_types=scratch,
    )


_sc_agg_deg = _make_sc_agg(True)
_sc_agg = _make_sc_agg(False)

BN = 1000


def _mm_body(x_ref, w_ref, o_ref):
    o_ref[...] = jnp.dot(x_ref[...], w_ref[...],
                         preferred_element_type=jnp.float32)


def _tc_matmul(x, w):
    n, d = x.shape
    h = w.shape[1]
    return pl.pallas_call(
        _mm_body,
        grid=(n // BN,),
        in_specs=[pl.BlockSpec((BN, d), lambda i: (i, 0)),
                  pl.BlockSpec((d, h), lambda i: (0, 0))],
        out_specs=pl.BlockSpec((BN, h), lambda i: (i, 0)),
        out_shape=jax.ShapeDtypeStruct((n, h), jnp.float32),
    )(x, w)


def _dinv(d_ref):
    deg = d_ref[:, 0:1] + d_ref[:, 1:2]
    return 1.0 / jnp.maximum(deg, 1.0)


def _layer_body(x_ref, ws_ref, b_ref, p_ref, d_ref, wn2_ref, h_ref, y2_ref):
    agg = (p_ref[0] + p_ref[1]) * _dinv(d_ref)
    h = jnp.dot(x_ref[...], ws_ref[...], preferred_element_type=jnp.float32)
    h = jnp.maximum(h + agg + b_ref[...], 0.0)
    h_ref[...] = h
    y2_ref[...] = jnp.dot(h, wn2_ref[...], preferred_element_type=jnp.float32)


def _tc_layer(x, w_self, b, part, degT, w_neigh2):
    return pl.pallas_call(
        _layer_body,
        grid=(N // BN,),
        in_specs=[pl.BlockSpec((BN, D), lambda i: (i, 0)),
                  pl.BlockSpec((D, H), lambda i: (0, 0)),
                  pl.BlockSpec((1, H), lambda i: (0, 0)),
                  pl.BlockSpec((NC, BN, H), lambda i: (0, i, 0)),
                  pl.BlockSpec((BN, NC), lambda i: (i, 0)),
                  pl.BlockSpec((H, H), lambda i: (0, 0))],
        out_specs=[pl.BlockSpec((BN, H), lambda i: (i, 0)),
                   pl.BlockSpec((BN, H), lambda i: (i, 0))],
        out_shape=[jax.ShapeDtypeStruct((N, H), jnp.float32),
                   jax.ShapeDtypeStruct((N, H), jnp.float32)],
    )(x, w_self, b, part, degT, w_neigh2)


def _final_body(h_ref, ws_ref, b_ref, q_ref, d_ref, wo_ref, bo_ref, o_ref):
    agg = (q_ref[0] + q_ref[1]) * _dinv(d_ref)
    h2 = jnp.dot(h_ref[...], ws_ref[...], preferred_element_type=jnp.float32)
    h2 = jnp.maximum(h2 + agg + b_ref[...], 0.0)
    o_ref[...] = jnp.dot(h2, wo_ref[...],
                         preferred_element_type=jnp.float32) + bo_ref[...]


def _tc_final(h1, w_self2, b2, part, degT, w_out, b_out):
    return pl.pallas_call(
        _final_body,
        grid=(N // BN,),
        in_specs=[pl.BlockSpec((BN, H), lambda i: (i, 0)),
                  pl.BlockSpec((H, H), lambda i: (0, 0)),
                  pl.BlockSpec((1, H), lambda i: (0, 0)),
                  pl.BlockSpec((NC, BN, H), lambda i: (0, i, 0)),
                  pl.BlockSpec((BN, NC), lambda i: (i, 0)),
                  pl.BlockSpec((H, C), lambda i: (0, 0)),
                  pl.BlockSpec((1, C), lambda i: (0, 0))],
        out_specs=pl.BlockSpec((BN, C), lambda i: (i, 0)),
        out_shape=jax.ShapeDtypeStruct((N, C), jnp.float32),
    )(h1, w_self2, b2, part, degT, w_out, b_out)


def kernel(features, edge_index, W_self1, W_neigh1, b1,
           W_self2, W_neigh2, b2, W_out, b_out):
    src4 = edge_index[0].reshape(NW, NSB, SB, CHUNK)
    dst4 = edge_index[1].reshape(NW, NSB, SB, CHUNK)
    z128 = jnp.zeros((NA, H), jnp.float32)
    z1 = jnp.zeros((NA,), jnp.float32)

    y1 = _tc_matmul(features, W_neigh1)
    part1, deg2 = _sc_agg_deg(y1, src4, dst4, z128, z1)
    degT = deg2.T
    h1, y2 = _tc_layer(features, W_self1, b1.reshape(1, H), part1, degT,
                       W_neigh2)
    (part2,) = _sc_agg(y2, src4, dst4, z128)
    out = _tc_final(h1, W_self2, b2.reshape(1, H), part2, degT,
                    W_out, b_out.reshape(1, C))
    return out
